# Initial kernel scaffold; baseline (speedup 1.0000x reference)
import functools

import jax
import jax.numpy as jnp
from jax import lax
from jax.experimental import pallas as pl
from jax.experimental.pallas import tpu as pltpu
from jax.experimental.pallas import tpu_sc as plsc

NC = 2
NS = 16
K = 128
NBUF = 4


def _ceil_to(x, m):
  return ((x + m - 1) // m) * m


def _embed_body(f_ref, w_ref, b_ref, o_ref):
  h = jnp.dot(f_ref[...], w_ref[...], preferred_element_type=jnp.float32)
  h = jnp.maximum(h + b_ref[...], 0.0)
  o_ref[0] = h[:, :32]
  o_ref[1] = h[:, 32:]


def _embed(feat, w, b, bn):
  n = feat.shape[0]
  fi = feat.shape[1]
  return pl.pallas_call(
      _embed_body,
      grid=(n // bn,),
      in_specs=[
          pl.BlockSpec((bn, fi), lambda i: (i, 0)),
          pl.BlockSpec((fi, 64), lambda i: (0, 0)),
          pl.BlockSpec((1, 64), lambda i: (0, 0)),
      ],
      out_specs=pl.BlockSpec((2, bn, 32), lambda i: (0, i, 0)),
      out_shape=jax.ShapeDtypeStruct((2, n, 32), jnp.float32),
  )(feat, w, b.reshape(1, 64))


def _update_body(h_ref, m_ref, w_ref, b_ref, o_ref):
  h = jnp.concatenate([h_ref[0], h_ref[1]], axis=-1)
  m = jnp.concatenate([m_ref[0], m_ref[1]], axis=-1)
  o = jnp.dot(m, w_ref[...], preferred_element_type=jnp.float32)
  o = jnp.maximum(h + o + b_ref[...], 0.0)
  o_ref[0] = o[:, :32]
  o_ref[1] = o[:, 32:]


def _update(h, msgs, w, b, bn):
  n = h.shape[1]
  return pl.pallas_call(
      _update_body,
      grid=(n // bn,),
      in_specs=[
          pl.BlockSpec((2, bn, 32), lambda i: (0, i, 0)),
          pl.BlockSpec((2, bn, 32), lambda i: (0, i, 0)),
          pl.BlockSpec((64, 64), lambda i: (0, 0)),
          pl.BlockSpec((1, 64), lambda i: (0, 0)),
      ],
      out_specs=pl.BlockSpec((2, bn, 32), lambda i: (0, i, 0)),
      out_shape=jax.ShapeDtypeStruct((2, n, 32), jnp.float32),
  )(h, msgs, w, b.reshape(1, 64))


def _score_body(h_ref, m_ref, w_ref, b_ref, ws_ref, bs_ref, o_ref):
  h = jnp.concatenate([h_ref[0], h_ref[1]], axis=-1)
  m = jnp.concatenate([m_ref[0], m_ref[1]], axis=-1)
  o = jnp.dot(m, w_ref[...], preferred_element_type=jnp.float32)
  o = jnp.maximum(h + o + b_ref[...], 0.0)
  o_ref[...] = jnp.dot(o, ws_ref[...], preferred_element_type=jnp.float32) + bs_ref[...]


def _score(h, msgs, w, b, ws, bs, bn):
  n = h.shape[1]
  return pl.pallas_call(
      _score_body,
      grid=(n // bn,),
      in_specs=[
          pl.BlockSpec((2, bn, 32), lambda i: (0, i, 0)),
          pl.BlockSpec((2, bn, 32), lambda i: (0, i, 0)),
          pl.BlockSpec((64, 64), lambda i: (0, 0)),
          pl.BlockSpec((1, 64), lambda i: (0, 0)),
          pl.BlockSpec((64, 1), lambda i: (0, 0)),
          pl.BlockSpec((1, 1), lambda i: (0, 0)),
      ],
      out_specs=pl.BlockSpec((bn, 1), lambda i: (i, 0)),
      out_shape=jax.ShapeDtypeStruct((n, 1), jnp.float32),
  )(h, msgs, w, b.reshape(1, 64), ws, bs.reshape(1, 1))


@functools.cache
def _make_sc_pass(n_src, n_dst_pad, nchunk_tot):
  del n_src
  nchunk_t = nchunk_tot // NS
  rows_per_tile = n_dst_pad // NS
  nz = rows_per_tile // K
  mesh = plsc.VectorSubcoreMesh(core_axis_name="c", subcore_axis_name="s")

  @functools.partial(
      pl.kernel,
      out_type=jax.ShapeDtypeStruct((NC, n_dst_pad, 32), jnp.float32),
      mesh=mesh,
      scratch_types=[
          pltpu.VMEM_SHARED((n_dst_pad, 32), jnp.float32),
          pltpu.VMEM((nchunk_t, K), jnp.int32),
          pltpu.VMEM((nchunk_t, K), jnp.int32),
          pltpu.VMEM((NBUF, K, 32), jnp.float32),
          pltpu.SemaphoreType.DMA,
          pltpu.SemaphoreType.DMA,
          pltpu.SemaphoreType.DMA,
          pltpu.SemaphoreType.DMA,
      ],
  )
  def sc_pass(t_hbm, sidx_hbm, didx_hbm, out_hbm, accum, sidx_v, didx_v,
              rows_v, sem0, sem1, sem2, sem3):
    sems = [sem0, sem1, sem2, sem3]
    c = lax.axis_index("c")
    s = lax.axis_index("s")

    def zf(i, carry):
      rows_v[0, i, pl.ds(0, 16)] = jnp.zeros((16,), jnp.float32)
      rows_v[0, i, pl.ds(16, 16)] = jnp.zeros((16,), jnp.float32)
      return carry
    lax.fori_loop(0, K, zf, 0)

    def zc(i, carry):
      pltpu.sync_copy(rows_v.at[0], accum.at[pl.ds((s * nz + i) * K, K)])
      return carry
    lax.fori_loop(0, nz, zc, 0)

    pltpu.sync_copy(sidx_hbm.at[pl.ds(s * nchunk_t, nchunk_t)], sidx_v)
    pltpu.sync_copy(didx_hbm.at[pl.ds(s * nchunk_t, nchunk_t)], didx_v)

    for b in range(NBUF):
      pltpu.async_copy(t_hbm.at[c].at[sidx_v.at[b]], rows_v.at[b], sems[b])

    plsc.subcore_barrier()

    def inner(jj, carry):
      for b in range(NBUF):
        j = jj * NBUF + b
        pltpu.make_async_copy(
            t_hbm.at[c].at[sidx_v.at[j]], rows_v.at[b], sems[b]).wait()
        pltpu.sync_copy(rows_v.at[b], accum.at[didx_v.at[j]], add=True)

        @pl.when(j + NBUF < nchunk_t)
        def _issue():
          pltpu.async_copy(
              t_hbm.at[c].at[sidx_v.at[j + NBUF]], rows_v.at[b], sems[b])
      return carry
    lax.fori_loop(0, nchunk_t // NBUF, inner, 0)

    plsc.subcore_barrier()
    pltpu.sync_copy(
        accum.at[pl.ds(s * rows_per_tile, rows_per_tile)],
        out_hbm.at[c].at[pl.ds(s * rows_per_tile, rows_per_tile)])

  return sc_pass


def kernel(var_feat, constr_feat, edge_index_var_to_constr,
           W_var, b_var, W_constr, b_constr,
           W_v2c, b_v2c, W_c2v, b_c2v, W_score, b_score):
  v = var_feat.shape[0]
  cn = constr_feat.shape[0]
  e = edge_index_var_to_constr.shape[1]

  v_pad = _ceil_to(v + 1, NS * K)
  c_pad = _ceil_to(cn + 1, NS * K)
  e_pad = _ceil_to(e, NS * K * NBUF)
  nchunk_tot = e_pad // K

  eidx = edge_index_var_to_constr.astype(jnp.int32)
  vidx, cidx = eidx[0], eidx[1]
  npad = e_pad - e
  ar = jnp.arange(npad, dtype=jnp.int32)
  sidx_v2c = jnp.concatenate([vidx, ar % v]).reshape(nchunk_tot, K)
  didx_v2c = jnp.concatenate([cidx, cn + ar % (c_pad - cn)]).reshape(nchunk_tot, K)
  sidx_c2v = jnp.concatenate([cidx, ar % cn]).reshape(nchunk_tot, K)
  didx_c2v = jnp.concatenate([vidx, v + ar % (v_pad - v)]).reshape(nchunk_tot, K)

  v2c = _make_sc_pass(v, c_pad, nchunk_tot)
  c2v = _make_sc_pass(cn, v_pad, nchunk_tot)

  h_var = _embed(var_feat, W_var, b_var, 1000)
  h_constr = _embed(constr_feat, W_constr, b_constr, 1000)

  rounds = 3
  for r in range(rounds):
    msgs_c = v2c(h_var, sidx_v2c, didx_v2c)
    h_constr = _update(h_constr, msgs_c, W_v2c, b_v2c, 1000)
    msgs_v = c2v(h_constr, sidx_c2v, didx_c2v)
    if r < rounds - 1:
      h_var = _update(h_var, msgs_v, W_c2v, b_c2v, 1000)
    else:
      scores = _score(h_var, msgs_v, W_c2v, b_c2v, W_score, b_score, 1000)

  return scores.reshape(-1)


# speedup vs baseline: 12.7992x; 12.7992x over previous
import functools

import jax
import jax.numpy as jnp
from jax import lax
from jax.experimental import pallas as pl
from jax.experimental.pallas import tpu as pltpu
from jax.experimental.pallas import tpu_sc as plsc

NC = 2
NS = 16
K = 128
NBUF = 4


def _ceil_to(x, m):
  return ((x + m - 1) // m) * m


def _embed_body(f_ref, w_ref, b_ref, o_ref):
  h = jnp.dot(f_ref[...], w_ref[...], preferred_element_type=jnp.float32)
  h = jnp.maximum(h + b_ref[...], 0.0)
  o_ref[0] = h[:, :32]
  o_ref[1] = h[:, 32:]


def _embed(feat, w, b, bn):
  n = feat.shape[0]
  fi = feat.shape[1]
  return pl.pallas_call(
      _embed_body,
      grid=(n // bn,),
      in_specs=[
          pl.BlockSpec((bn, fi), lambda i: (i, 0)),
          pl.BlockSpec((fi, 64), lambda i: (0, 0)),
          pl.BlockSpec((1, 64), lambda i: (0, 0)),
      ],
      out_specs=pl.BlockSpec((2, bn, 32), lambda i: (0, i, 0)),
      out_shape=jax.ShapeDtypeStruct((2, n, 32), jnp.float32),
  )(feat, w, b.reshape(1, 64))


def _update_body(h_ref, m_ref, w_ref, b_ref, o_ref):
  h = jnp.concatenate([h_ref[0], h_ref[1]], axis=-1)
  m = jnp.concatenate([m_ref[0], m_ref[1]], axis=-1)
  o = jnp.dot(m, w_ref[...], preferred_element_type=jnp.float32)
  o = jnp.maximum(h + o + b_ref[...], 0.0)
  o_ref[0] = o[:, :32]
  o_ref[1] = o[:, 32:]


def _update(h, msgs, w, b, bn):
  n = h.shape[1]
  return pl.pallas_call(
      _update_body,
      grid=(n // bn,),
      in_specs=[
          pl.BlockSpec((2, bn, 32), lambda i: (0, i, 0)),
          pl.BlockSpec((2, bn, 32), lambda i: (0, i, 0)),
          pl.BlockSpec((64, 64), lambda i: (0, 0)),
          pl.BlockSpec((1, 64), lambda i: (0, 0)),
      ],
      out_specs=pl.BlockSpec((2, bn, 32), lambda i: (0, i, 0)),
      out_shape=jax.ShapeDtypeStruct((2, n, 32), jnp.float32),
  )(h, msgs, w, b.reshape(1, 64))


def _score_body(h_ref, m_ref, w_ref, b_ref, ws_ref, bs_ref, o_ref):
  h = jnp.concatenate([h_ref[0], h_ref[1]], axis=-1)
  m = jnp.concatenate([m_ref[0], m_ref[1]], axis=-1)
  o = jnp.dot(m, w_ref[...], preferred_element_type=jnp.float32)
  o = jnp.maximum(h + o + b_ref[...], 0.0)
  o_ref[...] = jnp.dot(o, ws_ref[...], preferred_element_type=jnp.float32) + bs_ref[...]


def _score(h, msgs, w, b, ws, bs, bn):
  n = h.shape[1]
  return pl.pallas_call(
      _score_body,
      grid=(n // bn,),
      in_specs=[
          pl.BlockSpec((2, bn, 32), lambda i: (0, i, 0)),
          pl.BlockSpec((2, bn, 32), lambda i: (0, i, 0)),
          pl.BlockSpec((64, 64), lambda i: (0, 0)),
          pl.BlockSpec((1, 64), lambda i: (0, 0)),
          pl.BlockSpec((64, 1), lambda i: (0, 0)),
          pl.BlockSpec((1, 1), lambda i: (0, 0)),
      ],
      out_specs=pl.BlockSpec((bn, 1), lambda i: (i, 0)),
      out_shape=jax.ShapeDtypeStruct((n, 1), jnp.float32),
  )(h, msgs, w, b.reshape(1, 64), ws, bs.reshape(1, 1))


D = 8


@functools.cache
def _make_sc_pass(n_src, n_dst_pad, nchunk_tot):
  del n_src
  nchunk_t = nchunk_tot // NS
  rows_per_tile = n_dst_pad // NS
  nz = rows_per_tile // K
  assert nchunk_t % D == 0
  mesh = plsc.VectorSubcoreMesh(core_axis_name="c", subcore_axis_name="s")

  @functools.partial(
      pl.kernel,
      out_type=jax.ShapeDtypeStruct((NC, n_dst_pad, 32), jnp.float32),
      mesh=mesh,
      scratch_types=[
          pltpu.VMEM_SHARED((n_dst_pad, 32), jnp.float32),
          pltpu.VMEM((D, 2, K), jnp.int32),
          pltpu.VMEM((NBUF, K, 32), jnp.float32),
          [pltpu.SemaphoreType.DMA] * D,
          [pltpu.SemaphoreType.DMA] * NBUF,
      ],
      compiler_params=pltpu.CompilerParams(use_tc_tiling_on_sc=False),
  )
  def sc_pass(t_hbm, idx_hbm, out_hbm, accum, idx_v, rows_v, isem, gsem):
    c = lax.axis_index("c")
    s = lax.axis_index("s")
    row0 = s * nchunk_t

    def zf(i, carry):
      rows_v[0, i, pl.ds(0, 16)] = jnp.zeros((16,), jnp.float32)
      rows_v[0, i, pl.ds(16, 16)] = jnp.zeros((16,), jnp.float32)
      return carry
    lax.fori_loop(0, K, zf, 0)

    def zc(i, carry):
      pltpu.sync_copy(rows_v.at[0], accum.at[pl.ds((s * nz + i) * K, K)])
      return carry
    lax.fori_loop(0, nz, zc, 0)

    for u in range(D):
      pltpu.async_copy(idx_hbm.at[row0 + u], idx_v.at[u], isem[u])
    for u in range(NBUF):
      pltpu.make_async_copy(idx_hbm.at[row0 + u], idx_v.at[u], isem[u]).wait()
      pltpu.async_copy(t_hbm.at[c].at[idx_v.at[u].at[0]], rows_v.at[u], gsem[u])

    plsc.subcore_barrier()

    def inner(jj, carry):
      base = jj * D
      for u in range(D):
        j = base + u
        b = u % NBUF
        un = (u + NBUF) % D
        pltpu.make_async_copy(
            t_hbm.at[c].at[idx_v.at[u].at[0]], rows_v.at[b], gsem[b]).wait()
        pltpu.sync_copy(rows_v.at[b], accum.at[idx_v.at[u].at[1]], add=True)
        @pl.when(j + D < nchunk_t)
        def _refill():
          pltpu.async_copy(idx_hbm.at[row0 + j + D], idx_v.at[u], isem[u])
        @pl.when(j + NBUF < nchunk_t)
        def _issue():
          pltpu.make_async_copy(
              idx_hbm.at[row0 + j + NBUF], idx_v.at[un], isem[un]).wait()
          pltpu.async_copy(
              t_hbm.at[c].at[idx_v.at[un].at[0]], rows_v.at[b], gsem[b])
      return carry
    lax.fori_loop(0, nchunk_t // D, inner, 0)

    plsc.subcore_barrier()
    pltpu.sync_copy(
        accum.at[pl.ds(s * rows_per_tile, rows_per_tile)],
        out_hbm.at[c].at[pl.ds(s * rows_per_tile, rows_per_tile)])

  return sc_pass


def kernel(var_feat, constr_feat, edge_index_var_to_constr,
           W_var, b_var, W_constr, b_constr,
           W_v2c, b_v2c, W_c2v, b_c2v, W_score, b_score):
  v = var_feat.shape[0]
  cn = constr_feat.shape[0]
  e = edge_index_var_to_constr.shape[1]

  v_pad = _ceil_to(v + 1, NS * K)
  c_pad = _ceil_to(cn + 1, NS * K)
  e_pad = _ceil_to(e, NS * K * D)
  nchunk_tot = e_pad // K

  eidx = edge_index_var_to_constr.astype(jnp.int32)
  vidx, cidx = eidx[0], eidx[1]
  npad = e_pad - e
  ar = jnp.arange(npad, dtype=jnp.int32)
  sidx_v2c = jnp.concatenate([vidx, ar % v]).reshape(nchunk_tot, 1, K)
  didx_v2c = jnp.concatenate([cidx, cn + ar % (c_pad - cn)]).reshape(nchunk_tot, 1, K)
  sidx_c2v = jnp.concatenate([cidx, ar % cn]).reshape(nchunk_tot, 1, K)
  didx_c2v = jnp.concatenate([vidx, v + ar % (v_pad - v)]).reshape(nchunk_tot, 1, K)
  idx_v2c = jnp.concatenate([sidx_v2c, didx_v2c], axis=1)
  idx_c2v = jnp.concatenate([sidx_c2v, didx_c2v], axis=1)

  v2c = _make_sc_pass(v, c_pad, nchunk_tot)
  c2v = _make_sc_pass(cn, v_pad, nchunk_tot)

  h_var = _embed(var_feat, W_var, b_var, 1000)
  h_constr = _embed(constr_feat, W_constr, b_constr, 1000)

  rounds = 3
  for r in range(rounds):
    msgs_c = v2c(h_var, idx_v2c)
    h_constr = _update(h_constr, msgs_c, W_v2c, b_v2c, 1000)
    msgs_v = c2v(h_constr, idx_c2v)
    if r < rounds - 1:
      h_var = _update(h_var, msgs_v, W_c2v, b_c2v, 1000)
    else:
      scores = _score(h_var, msgs_v, W_c2v, b_c2v, W_score, b_score, 1000)

  return scores.reshape(-1)
